# baseline (speedup 1.0000x reference)
import functools

import numpy as np

import jax
import jax.numpy as jnp
from jax import lax
from jax.experimental import pallas as pl
from jax.experimental.pallas import tpu as pltpu
from jax.experimental.pallas import tpu_sc as plsc

H = 128
NC, NS = 2, 16
NW = NC * NS
CHUNK = 64
RB = 256


def _ln(x, g, b):
    mu = jnp.mean(x, -1, keepdims=True)
    var = jnp.mean((x - mu) ** 2, -1, keepdims=True)
    return (x - mu) / jnp.sqrt(var + 1e-5) * g + b


def _relu(x):
    return jnp.maximum(x, 0.0)


def _dot(a, b):
    return jnp.dot(a, b, preferred_element_type=jnp.float32)


def _full(shape):
    return pl.BlockSpec(shape, lambda i: (0,) * len(shape))


def _pre_body(x_ref, w1, b1, g1, be1, w2, b2, g2, be2, o_ref):
    x = x_ref[...]
    h = _relu(_ln(_dot(x, w1[...]) + b1[...], g1[...], be1[...]))
    o_ref[...] = _relu(_ln(_dot(h, w2[...]) + b2[...], g2[...], be2[...]))


def _pre_call(x, p):
    n = x.shape[0]
    ws = [p["W1"], p["b1"], p["g1"], p["be1"], p["W2"], p["b2"], p["g2"], p["be2"]]
    ws = [w.reshape(1, H) if w.ndim == 1 else w for w in ws]
    specs = [pl.BlockSpec((RB, H), lambda i: (i, 0))]
    specs += [_full(tuple(w.shape)) for w in ws]
    return pl.pallas_call(
        _pre_body,
        grid=(n // RB,),
        in_specs=specs,
        out_specs=pl.BlockSpec((RB, H), lambda i: (i, 0)),
        out_shape=jax.ShapeDtypeStruct((n, H), jnp.float32),
    )(x, *ws)


def _mm_body(x_ref, w_ref, o_ref):
    o_ref[...] = _dot(x_ref[...], w_ref[...])


def _mm_call(x, w):
    n, k = x.shape[0], w.shape[1]
    return pl.pallas_call(
        _mm_body,
        grid=(n // RB,),
        in_specs=[pl.BlockSpec((RB, H), lambda i: (i, 0)), _full((H, k))],
        out_specs=pl.BlockSpec((RB, k), lambda i: (i, 0)),
        out_shape=jax.ShapeDtypeStruct((n, k), jnp.float32),
    )(x, w)


def _fin_part(num_ref, den_ref, b, g, be):
    n = num_ref[0] + num_ref[1]
    d = jnp.sum(den_ref[...], axis=(0, 1))
    y = n / (d[:, None] + 1e-16) + b[...]
    return _relu(_ln(y, g[...], be[...]))


def _fin1_body(num_ref, den_ref, b, g, be, o_ref):
    o_ref[...] = _fin_part(num_ref, den_ref, b, g, be)


def _fin2_body(na, da, ba, ga, bea, nb, db, bb, gb, beb, o_ref):
    o_ref[...] = (_fin_part(na, da, ba, ga, bea)
                  + _fin_part(nb, db, bb, gb, beb))


def _fin_specs(n):
    return [
        pl.BlockSpec((NC, RB, H), lambda i: (0, i, 0)),
        pl.BlockSpec((NC, NS, RB), lambda i: (0, 0, i)),
        _full((1, H)), _full((1, H)), _full((1, H)),
    ]


def _fin1_call(num, den, p):
    n = num.shape[1]
    args = [num, den, p["b"].reshape(1, H), p["ln_g"].reshape(1, H),
            p["ln_b"].reshape(1, H)]
    return pl.pallas_call(
        _fin1_body,
        grid=(n // RB,),
        in_specs=_fin_specs(n),
        out_specs=pl.BlockSpec((RB, H), lambda i: (i, 0)),
        out_shape=jax.ShapeDtypeStruct((n, H), jnp.float32),
    )(*args)


def _fin2_call(numa, dena, pa, numb, denb, pb):
    n = numa.shape[1]
    args = [numa, dena, pa["b"].reshape(1, H), pa["ln_g"].reshape(1, H),
            pa["ln_b"].reshape(1, H),
            numb, denb, pb["b"].reshape(1, H), pb["ln_g"].reshape(1, H),
            pb["ln_b"].reshape(1, H)]
    return pl.pallas_call(
        _fin2_body,
        grid=(n // RB,),
        in_specs=_fin_specs(n) + _fin_specs(n),
        out_specs=pl.BlockSpec((RB, H), lambda i: (i, 0)),
        out_shape=jax.ShapeDtypeStruct((n, H), jnp.float32),
    )(*args)


def _agg_body(x_ref, oh_ref, gw1, gb1, gw2, gb2, tw, tb, num_ref, den_ref):
    i = pl.program_id(0)
    x = x_ref[...]
    h = _relu(_dot(x, gw1[...]) + gb1[...])
    gate = _dot(h, gw2[...]) + gb2[...]
    w = jnp.exp(gate[:, 0:1])
    t = _relu(_dot(x, tw[...]) + tb[...])
    cn = lax.dot_general(oh_ref[...], w * t, (((0,), (0,)), ((), ())),
                         preferred_element_type=jnp.float32)
    cd = lax.dot_general(oh_ref[...], jnp.broadcast_to(w, t.shape),
                         (((0,), (0,)), ((), ())),
                         preferred_element_type=jnp.float32)

    @pl.when(i == 0)
    def _():
        num_ref[...] = cn
        den_ref[...] = cd

    @pl.when(i > 0)
    def _():
        num_ref[...] = num_ref[...] + cn
        den_ref[...] = den_ref[...] + cd


def _agg_call(x, oh, ag):
    n = x.shape[0]
    gw1 = jnp.zeros((H, H), jnp.float32).at[:, : H // 2].set(ag["gW1"])
    gb1 = jnp.zeros((1, H), jnp.float32).at[:, : H // 2].set(ag["gb1"])
    gw2 = jnp.zeros((H, H), jnp.float32).at[: H // 2, 0:1].set(ag["gW2"])
    gb2 = jnp.zeros((1, H), jnp.float32).at[:, 0:1].set(ag["gb2"])
    tb = ag["tb"].reshape(1, H)
    return pl.pallas_call(
        _agg_body,
        grid=(n // RB,),
        in_specs=[pl.BlockSpec((RB, H), lambda i: (i, 0)),
                  pl.BlockSpec((RB, H), lambda i: (i, 0)),
                  _full((H, H)), _full((1, H)), _full((H, H)), _full((1, H)),
                  _full((H, H)), _full((1, H))],
        out_specs=[_full((H, H)), _full((H, H))],
        out_shape=[jax.ShapeDtypeStruct((H, H), jnp.float32),
                   jax.ShapeDtypeStruct((H, H), jnp.float32)],
    )(x, oh, gw1, gb1, gw2, gb2, ag["tW"], tb)


def _head_body(num_ref, den_ref, w, b, o_ref):
    pooled = num_ref[...] / (den_ref[...] + 1e-16)
    o_ref[...] = _dot(pooled, w[...]) + b[...]


def _head_call(num, den, hd):
    w = jnp.zeros((H, H), jnp.float32).at[:, :2].set(hd["W"])
    b = jnp.zeros((1, H), jnp.float32).at[:, :2].set(hd["b"])
    return pl.pallas_call(
        _head_body,
        grid=(1,),
        in_specs=[_full((H, H)), _full((H, H)), _full((H, H)), _full((1, H))],
        out_specs=_full((H, H)),
        out_shape=jax.ShapeDtypeStruct((H, H), jnp.float32),
    )(num, den, w, b)


_GDN = lax.GatherDimensionNumbers(offset_dims=(), collapsed_slice_dims=(0,),
                                  start_index_map=(0,))

_PERM = np.arange(H).reshape(H // 32, 16, 2).transpose(0, 2, 1).reshape(H)
_INVPERM = np.argsort(_PERM)


def _hsum(a, perms):
    for p in perms:
        g = lax.gather(a, p, _GDN, (1,),
                       mode=lax.GatherScatterMode.PROMISE_IN_BOUNDS)
        a = a + g
    return a


@functools.lru_cache(maxsize=None)
def _edge_kernel(n_src, n_dst, e_pad, has_attr):
    e_tile = e_pad // NW
    n_chunks = e_tile // CHUNK
    rows_tile = n_dst // NS
    n_zcopy = rows_tile // CHUNK
    mesh = plsc.VectorSubcoreMesh(core_axis_name="c", subcore_axis_name="s")

    scratch = [
        pltpu.VMEM((CHUNK,), jnp.int32),
        pltpu.VMEM((CHUNK,), jnp.int32),
        pltpu.VMEM((CHUNK,), jnp.int32),
        pltpu.VMEM((CHUNK,), jnp.int32),
        pltpu.VMEM((CHUNK, H // 2), jnp.int32),
        pltpu.VMEM((CHUNK, H // 2), jnp.int32),
        pltpu.VMEM((CHUNK, H // 2), jnp.int32),
        pltpu.VMEM((CHUNK, H // 2), jnp.int32),
        pltpu.VMEM((CHUNK, H), jnp.float32),
        pltpu.VMEM((CHUNK, H), jnp.float32),
        pltpu.VMEM((CHUNK,), jnp.float32),
        pltpu.VMEM((CHUNK,), jnp.float32),
        pltpu.VMEM((n_dst,), jnp.float32),
        pltpu.VMEM((H,), jnp.float32),
        pltpu.VMEM((H,), jnp.float32),
        pltpu.VMEM_SHARED((n_dst, H), jnp.float32),
        pltpu.SemaphoreType.DMA,
        pltpu.SemaphoreType.DMA,
        pltpu.SemaphoreType.DMA,
        pltpu.SemaphoreType.DMA,
        pltpu.SemaphoreType.DMA,
        pltpu.SemaphoreType.DMA,
    ]
    out_type = [jax.ShapeDtypeStruct((NC, n_dst, H), jnp.float32),
                jax.ShapeDtypeStruct((NC, NS, n_dst), jnp.float32)]

    @functools.partial(pl.kernel, mesh=mesh, out_type=out_type,
                       scratch_types=scratch,
                       compiler_params=pltpu.CompilerParams(
                           needs_layout_passes=False,
                           use_tc_tiling_on_sc=False))
    def k(xl_hbm, xr_hbm, src_hbm, dst_hbm, att_hbm, attr_hbm, we_hbm,
          num_out, den_out,
          srcv0, srcv1, dstv0, dstv1, xlr0, xlr1, xrr0, xrr1,
          scf0, scf1, attrv0, attrv1, denl, attv, wev, numsh,
          isem0, isem1, gsem0, gsem1, ssem0, ssem1):
        c = lax.axis_index("c")
        s = lax.axis_index("s")
        wid = c * NS + s
        zero16 = jnp.zeros((16,), jnp.float32)
        srcv = [srcv0, srcv1]
        dstv = [dstv0, dstv1]
        xlr = [xlr0, xlr1]
        xrr = [xrr0, xrr1]
        scf = [scf0, scf1]
        attrv = [attrv0, attrv1]
        isem = [isem0, isem1]
        gsem = [gsem0, gsem1]
        ssem = [ssem0, ssem1]
        ebase = wid * e_tile

        def _idx_copies(ci, b):
            cb = ebase + ci * CHUNK
            cps = [pltpu.make_async_copy(src_hbm.at[pl.ds(cb, CHUNK)],
                                         srcv[b], isem[b]),
                   pltpu.make_async_copy(dst_hbm.at[pl.ds(cb, CHUNK)],
                                         dstv[b], isem[b])]
            if has_attr:
                cps.append(pltpu.make_async_copy(
                    attr_hbm.at[pl.ds(cb, CHUNK)], attrv[b], isem[b]))
            return cps

        def _gath_copies(b):
            return [pltpu.make_async_copy(xl_hbm.at[srcv[b]], xlr[b], gsem[b]),
                    pltpu.make_async_copy(xr_hbm.at[dstv[b]], xrr[b], gsem[b])]

        def _scat(b):
            return pltpu.make_async_copy(scf[b], numsh.at[dstv[b]], ssem[b])

        def _zb(i, _):
            for j in range(H // 16):
                scf0[i, pl.ds(j * 16, 16)] = zero16
            return 0
        lax.fori_loop(0, CHUNK, _zb, 0)

        def _zd(i, _):
            denl[pl.ds(i * 16, 16)] = zero16
            return 0
        lax.fori_loop(0, n_dst // 16, _zd, 0)

        base_rows = s * rows_tile
        for q in range(n_zcopy):
            pltpu.sync_copy(scf0, numsh.at[pl.ds(base_rows + q * CHUNK, CHUNK)])
        plsc.subcore_barrier()

        pltpu.sync_copy(att_hbm, attv)
        if has_attr:
            pltpu.sync_copy(we_hbm, wev)
        att_j = [attv[pl.ds(j * 16, 16)] for j in range(H // 16)]
        we_j = ([wev[pl.ds(j * 16, 16)] for j in range(H // 16)]
                if has_attr else None)

        for cp in _idx_copies(0, 0):
            cp.start()
        for cp in _idx_copies(1, 1):
            cp.start()
        for cp in _idx_copies(0, 0):
            cp.wait()
        for cp in _gath_copies(0):
            cp.start()

        def _chunk_body(ci, cur):
            nxt = 1 - cur
            for cp in _gath_copies(cur):
                cp.wait()
            @pl.when(ci + 2 < n_chunks)
            def _():
                for cp in _idx_copies(ci + 2, cur):
                    cp.start()
            @pl.when(ci + 1 < n_chunks)
            def _():
                @pl.when(ci >= 1)
                def _():
                    _scat(nxt).wait()
                for cp in _idx_copies(ci + 1, nxt):
                    cp.wait()
                for cp in _gath_copies(nxt):
                    cp.start()
            lanes = lax.iota(jnp.int32, 16)
            perms = [(lanes ^ sh).reshape(16, 1) for sh in (8, 4, 2, 1)]
            xl_b, xr_b, sc_b = xlr[cur], xrr[cur], scf[cur]
            at_b, ds_b = attrv[cur], dstv[cur]

            def _unp(v):
                return plsc.unpack(plsc.bitcast(v, jnp.bfloat16),
                                   format=plsc.PackFormat.INTERLEAVED)

            def _dotl(kk, _):
                evv = zero16
                if has_attr:
                    av = at_b[pl.ds(kk * 16, 16)]
                for t in range(16):
                    i = kk * 16 + t
                    acc = zero16
                    for j in range(H // 32):
                        xa, xb = _unp(xl_b[i, pl.ds(j * 16, 16)])
                        ra, rb = _unp(xr_b[i, pl.ds(j * 16, 16)])
                        ma = xa + ra
                        mb = xb + rb
                        if has_attr:
                            ma = ma + av[t] * we_j[2 * j]
                            mb = mb + av[t] * we_j[2 * j + 1]
                        ma = jnp.maximum(ma, ma * 0.2)
                        mb = jnp.maximum(mb, mb * 0.2)
                        acc = acc + ma * att_j[2 * j] + mb * att_j[2 * j + 1]
                    evv = jnp.where(lanes == t, _hsum(acc, perms), evv)
                vv = jnp.exp(evv)
                plsc.addupdate_scatter(denl, [ds_b[pl.ds(kk * 16, 16)]], vv)
                for t in range(16):
                    i = kk * 16 + t
                    sc = vv[t]
                    for j in range(H // 32):
                        xa, xb = _unp(xl_b[i, pl.ds(j * 16, 16)])
                        sc_b[i, pl.ds(j * 32, 16)] = xa * sc
                        sc_b[i, pl.ds(j * 32 + 16, 16)] = xb * sc
                return 0
            lax.fori_loop(0, CHUNK // 16, _dotl, 0)
            _scat(cur).start(add=True)

        def _chunk(ci, _):
            @pl.when(ci % 2 == 0)
            def _():
                _chunk_body(ci, 0)

            @pl.when(ci % 2 == 1)
            def _():
                _chunk_body(ci, 1)
            return 0
        lax.fori_loop(0, n_chunks, _chunk, 0)

        _scat(0).wait()
        _scat(1).wait()

        plsc.subcore_barrier()
        pltpu.sync_copy(denl, den_out.at[c, s])
        for q in range(n_zcopy):
            r0 = base_rows + q * CHUNK
            pltpu.sync_copy(numsh.at[pl.ds(r0, CHUNK)],
                            num_out.at[c].at[pl.ds(r0, CHUNK)])

    return k


def _pad_rows(x, n):
    return jnp.pad(x, ((0, n - x.shape[0]), (0, 0)))


def _pad_edges(src, dst, e_pad, dst_sentinel, attr=None):
    e = src.shape[0]
    src = jnp.pad(src, (0, e_pad - e))
    dst = jnp.pad(dst, (0, e_pad - e), constant_values=dst_sentinel)
    if attr is not None:
        attr = jnp.pad(attr, (0, e_pad - e))
    return src, dst, attr


def _edge_pass(xl, xr, src, dst, att, attr=None, we=None):
    n_src, n_dst = xl.shape[0], xr.shape[0]
    e_pad = src.shape[0]
    has_attr = attr is not None
    if not has_attr:
        attr = jnp.zeros((e_pad,), jnp.float32)
        we = jnp.zeros((H,), jnp.float32)
    perm = jnp.asarray(_PERM)

    def _asi32(x):
        xb = x.astype(jnp.bfloat16).reshape(x.shape[0], H // 2, 2)
        return lax.bitcast_convert_type(xb, jnp.int32)

    k = _edge_kernel(n_src, n_dst, e_pad, has_attr)
    num, den = k(_asi32(xl), _asi32(xr), src, dst, att[perm], attr, we[perm])
    num = jnp.take(num, jnp.asarray(_INVPERM), axis=2)
    return num, den


def _round_up(x, m):
    return (x + m - 1) // m * m


def kernel(params, gene_idx, reaction_idx, metabolite_idx, edge_index_ppi,
           edge_index_reg, edge_index_gpr, edge_index_rmr, edge_attr_rmr,
           gene_batch):
    gn = gene_idx.shape[0]
    rn = reaction_idx.shape[0]
    mn = metabolite_idx.shape[0]
    ngp = _round_up(gn, 1024)
    nrp = _round_up(rn, 1024)
    nmp = _round_up(mn, 1024)

    x_g = _pre_call(_pad_rows(params["emb_gene"], ngp), params["pre"])
    x_r = _pad_rows(params["emb_rxn"], nrp)
    x_m = _pad_rows(params["emb_met"], nmp)

    unit = NW * CHUNK
    e_ppi_pad = _round_up(edge_index_ppi.shape[1], unit)
    e_gpr_pad = _round_up(edge_index_gpr.shape[1], unit)
    src_ppi, dst_ppi, _ = _pad_edges(edge_index_ppi[0], edge_index_ppi[1],
                                     e_ppi_pad, gn)
    src_reg, dst_reg, _ = _pad_edges(edge_index_reg[0], edge_index_reg[1],
                                     e_ppi_pad, gn)
    src_gpr, dst_gpr, _ = _pad_edges(edge_index_gpr[0], edge_index_gpr[1],
                                     e_gpr_pad, rn)
    src_rmr, dst_rmr, attr_rmr = _pad_edges(
        edge_index_rmr[0], edge_index_rmr[1], e_gpr_pad, mn,
        edge_attr_rmr[:, 0])

    for lp in params["layers"]:
        wg = jnp.concatenate([lp["ppi"]["Wl"], lp["ppi"]["Wr"],
                              lp["reg"]["Wl"], lp["reg"]["Wr"],
                              lp["gpr"]["Wl"]], axis=1)
        pg = _mm_call(x_g, wg)
        xl_ppi, xr_ppi = pg[:, 0:H], pg[:, H:2 * H]
        xl_reg, xr_reg = pg[:, 2 * H:3 * H], pg[:, 3 * H:4 * H]
        xl_gpr = pg[:, 4 * H:5 * H]
        wr = jnp.concatenate([lp["gpr"]["Wr"], lp["rmr"]["Wl"]], axis=1)
        pr = _mm_call(x_r, wr)
        xr_gpr, xl_rmr = pr[:, 0:H], pr[:, H:2 * H]
        xr_rmr = _mm_call(x_m, lp["rmr"]["Wr"])

        num_p, den_p = _edge_pass(xl_ppi, xr_ppi, src_ppi, dst_ppi,
                                  lp["ppi"]["att"])
        num_r, den_r = _edge_pass(xl_reg, xr_reg, src_reg, dst_reg,
                                  lp["reg"]["att"])
        num_q, den_q = _edge_pass(xl_gpr, xr_gpr, src_gpr, dst_gpr,
                                  lp["gpr"]["att"])
        num_m, den_m = _edge_pass(xl_rmr, xr_rmr, src_rmr, dst_rmr,
                                  lp["rmr"]["att"], attr_rmr,
                                  lp["rmr"]["We"].reshape(H))

        x_g = _fin2_call(num_p, den_p, lp["ppi"], num_r, den_r, lp["reg"])
        x_r = _fin1_call(num_q, den_q, lp["gpr"])
        x_m = _fin1_call(num_m, den_m, lp["rmr"])

    batch_pad = jnp.pad(gene_batch, (0, ngp - gn), constant_values=4)
    oh = jax.nn.one_hot(batch_pad, H, dtype=jnp.float32)
    num_pool, den_pool = _agg_call(x_g, oh, params["agg"])
    out = _head_call(num_pool, den_pool, params["head"])
    return out[:4, :2]


# speedup vs baseline: 1.5522x; 1.5522x over previous
import functools

import jax
import jax.numpy as jnp
from jax import lax
from jax.experimental import pallas as pl
from jax.experimental.pallas import tpu as pltpu
from jax.experimental.pallas import tpu_sc as plsc

H = 128
NC, NS = 2, 16
NW = NC * NS
CHUNK = 64
RB = 256


def _ln(x, g, b):
    mu = jnp.mean(x, -1, keepdims=True)
    var = jnp.mean((x - mu) ** 2, -1, keepdims=True)
    return (x - mu) / jnp.sqrt(var + 1e-5) * g + b


def _relu(x):
    return jnp.maximum(x, 0.0)


def _dot(a, b):
    return jnp.dot(a, b, preferred_element_type=jnp.float32)


def _full(shape):
    return pl.BlockSpec(shape, lambda i: (0,) * len(shape))


def _pre_body(x_ref, w1, b1, g1, be1, w2, b2, g2, be2, o_ref):
    x = x_ref[...]
    h = _relu(_ln(_dot(x, w1[...]) + b1[...], g1[...], be1[...]))
    o_ref[...] = _relu(_ln(_dot(h, w2[...]) + b2[...], g2[...], be2[...]))


def _pre_call(x, p):
    n = x.shape[0]
    ws = [p["W1"], p["b1"], p["g1"], p["be1"], p["W2"], p["b2"], p["g2"], p["be2"]]
    ws = [w.reshape(1, H) if w.ndim == 1 else w for w in ws]
    specs = [pl.BlockSpec((RB, H), lambda i: (i, 0))]
    specs += [_full(tuple(w.shape)) for w in ws]
    return pl.pallas_call(
        _pre_body,
        grid=(n // RB,),
        in_specs=specs,
        out_specs=pl.BlockSpec((RB, H), lambda i: (i, 0)),
        out_shape=jax.ShapeDtypeStruct((n, H), jnp.float32),
    )(x, *ws)


def _mm_body(x_ref, w_ref, o_ref):
    o_ref[...] = _dot(x_ref[...], w_ref[...])


def _mm_call(x, w):
    n, k = x.shape[0], w.shape[1]
    return pl.pallas_call(
        _mm_body,
        grid=(n // RB,),
        in_specs=[pl.BlockSpec((RB, H), lambda i: (i, 0)), _full((H, k))],
        out_specs=pl.BlockSpec((RB, k), lambda i: (i, 0)),
        out_shape=jax.ShapeDtypeStruct((n, k), jnp.float32),
    )(x, w)


def _fin_part(num_ref, den_ref, b, g, be):
    n = num_ref[0] + num_ref[1]
    d = jnp.sum(den_ref[...], axis=(0, 1))
    y = n / (d[:, None] + 1e-16) + b[...]
    return _relu(_ln(y, g[...], be[...]))


def _fin1_body(num_ref, den_ref, b, g, be, o_ref):
    o_ref[...] = _fin_part(num_ref, den_ref, b, g, be)


def _fin2_body(na, da, ba, ga, bea, nb, db, bb, gb, beb, o_ref):
    o_ref[...] = (_fin_part(na, da, ba, ga, bea)
                  + _fin_part(nb, db, bb, gb, beb))


def _fin_specs(n):
    return [
        pl.BlockSpec((NC, RB, H), lambda i: (0, i, 0)),
        pl.BlockSpec((NC, NS, RB), lambda i: (0, 0, i)),
        _full((1, H)), _full((1, H)), _full((1, H)),
    ]


def _fin1_call(num, den, p):
    n = num.shape[1]
    args = [num, den, p["b"].reshape(1, H), p["ln_g"].reshape(1, H),
            p["ln_b"].reshape(1, H)]
    return pl.pallas_call(
        _fin1_body,
        grid=(n // RB,),
        in_specs=_fin_specs(n),
        out_specs=pl.BlockSpec((RB, H), lambda i: (i, 0)),
        out_shape=jax.ShapeDtypeStruct((n, H), jnp.float32),
    )(*args)


def _fin2_call(numa, dena, pa, numb, denb, pb):
    n = numa.shape[1]
    args = [numa, dena, pa["b"].reshape(1, H), pa["ln_g"].reshape(1, H),
            pa["ln_b"].reshape(1, H),
            numb, denb, pb["b"].reshape(1, H), pb["ln_g"].reshape(1, H),
            pb["ln_b"].reshape(1, H)]
    return pl.pallas_call(
        _fin2_body,
        grid=(n // RB,),
        in_specs=_fin_specs(n) + _fin_specs(n),
        out_specs=pl.BlockSpec((RB, H), lambda i: (i, 0)),
        out_shape=jax.ShapeDtypeStruct((n, H), jnp.float32),
    )(*args)


def _agg_body(x_ref, oh_ref, gw1, gb1, gw2, gb2, tw, tb, num_ref, den_ref):
    i = pl.program_id(0)
    x = x_ref[...]
    h = _relu(_dot(x, gw1[...]) + gb1[...])
    gate = _dot(h, gw2[...]) + gb2[...]
    w = jnp.exp(gate[:, 0:1])
    t = _relu(_dot(x, tw[...]) + tb[...])
    cn = lax.dot_general(oh_ref[...], w * t, (((0,), (0,)), ((), ())),
                         preferred_element_type=jnp.float32)
    cd = lax.dot_general(oh_ref[...], jnp.broadcast_to(w, t.shape),
                         (((0,), (0,)), ((), ())),
                         preferred_element_type=jnp.float32)

    @pl.when(i == 0)
    def _():
        num_ref[...] = cn
        den_ref[...] = cd

    @pl.when(i > 0)
    def _():
        num_ref[...] = num_ref[...] + cn
        den_ref[...] = den_ref[...] + cd


def _agg_call(x, oh, ag):
    n = x.shape[0]
    gw1 = jnp.zeros((H, H), jnp.float32).at[:, : H // 2].set(ag["gW1"])
    gb1 = jnp.zeros((1, H), jnp.float32).at[:, : H // 2].set(ag["gb1"])
    gw2 = jnp.zeros((H, H), jnp.float32).at[: H // 2, 0:1].set(ag["gW2"])
    gb2 = jnp.zeros((1, H), jnp.float32).at[:, 0:1].set(ag["gb2"])
    tb = ag["tb"].reshape(1, H)
    return pl.pallas_call(
        _agg_body,
        grid=(n // RB,),
        in_specs=[pl.BlockSpec((RB, H), lambda i: (i, 0)),
                  pl.BlockSpec((RB, H), lambda i: (i, 0)),
                  _full((H, H)), _full((1, H)), _full((H, H)), _full((1, H)),
                  _full((H, H)), _full((1, H))],
        out_specs=[_full((H, H)), _full((H, H))],
        out_shape=[jax.ShapeDtypeStruct((H, H), jnp.float32),
                   jax.ShapeDtypeStruct((H, H), jnp.float32)],
    )(x, oh, gw1, gb1, gw2, gb2, ag["tW"], tb)


def _head_body(num_ref, den_ref, w, b, o_ref):
    pooled = num_ref[...] / (den_ref[...] + 1e-16)
    o_ref[...] = _dot(pooled, w[...]) + b[...]


def _head_call(num, den, hd):
    w = jnp.zeros((H, H), jnp.float32).at[:, :2].set(hd["W"])
    b = jnp.zeros((1, H), jnp.float32).at[:, :2].set(hd["b"])
    return pl.pallas_call(
        _head_body,
        grid=(1,),
        in_specs=[_full((H, H)), _full((H, H)), _full((H, H)), _full((1, H))],
        out_specs=_full((H, H)),
        out_shape=jax.ShapeDtypeStruct((H, H), jnp.float32),
    )(num, den, w, b)


_GDN = lax.GatherDimensionNumbers(offset_dims=(), collapsed_slice_dims=(0,),
                                  start_index_map=(0,))


def _hsum(a, perms):
    for p in perms:
        g = lax.gather(a, p, _GDN, (1,),
                       mode=lax.GatherScatterMode.PROMISE_IN_BOUNDS)
        a = a + g
    return a


@functools.lru_cache(maxsize=None)
def _edge_kernel(n_src, n_dst, e_pad, has_attr):
    e_tile = e_pad // NW
    n_chunks = e_tile // CHUNK
    rows_tile = n_dst // NS
    n_zcopy = rows_tile // CHUNK
    mesh = plsc.VectorSubcoreMesh(core_axis_name="c", subcore_axis_name="s")

    scratch = [
        pltpu.VMEM((CHUNK,), jnp.int32),
        pltpu.VMEM((CHUNK,), jnp.int32),
        pltpu.VMEM((CHUNK,), jnp.int32),
        pltpu.VMEM((CHUNK,), jnp.int32),
        pltpu.VMEM((CHUNK, H), jnp.float32),
        pltpu.VMEM((CHUNK, H), jnp.float32),
        pltpu.VMEM((CHUNK, H), jnp.float32),
        pltpu.VMEM((CHUNK, H), jnp.float32),
        pltpu.VMEM((CHUNK,), jnp.float32),
        pltpu.VMEM((CHUNK,), jnp.float32),
        pltpu.VMEM((CHUNK,), jnp.int32),
        pltpu.VMEM((CHUNK,), jnp.int32),
        pltpu.VMEM((CHUNK,), jnp.float32),
        pltpu.VMEM((CHUNK,), jnp.float32),
        pltpu.VMEM((n_dst,), jnp.float32),
        pltpu.VMEM((H,), jnp.float32),
        pltpu.VMEM((H,), jnp.float32),
        pltpu.VMEM_SHARED((n_dst, H), jnp.float32),
        pltpu.SemaphoreType.DMA,
        pltpu.SemaphoreType.DMA,
        pltpu.SemaphoreType.DMA,
        pltpu.SemaphoreType.DMA,
        pltpu.SemaphoreType.DMA,
        pltpu.SemaphoreType.DMA,
    ]
    out_type = [jax.ShapeDtypeStruct((NC, n_dst, H), jnp.float32),
                jax.ShapeDtypeStruct((NC, NS, n_dst), jnp.float32)]

    @functools.partial(pl.kernel, mesh=mesh, out_type=out_type,
                       scratch_types=scratch,
                       compiler_params=pltpu.CompilerParams(
                           needs_layout_passes=False))
    def k(xl_hbm, xr_hbm, src_hbm, dst_hbm, att_hbm, attr_hbm, we_hbm,
          num_out, den_out,
          srcv0, srcv1, dstv0, dstv1, xlr0, xlr1, xrr0, xrr1,
          attrv0, attrv1, dsc0, dsc1, asc0, asc1, denl, attv, wev, numsh,
          isem0, isem1, gsem0, gsem1, ssem0, ssem1):
        c = lax.axis_index("c")
        s = lax.axis_index("s")
        wid = c * NS + s
        zero16 = jnp.zeros((16,), jnp.float32)
        srcv = [srcv0, srcv1]
        dstv = [dstv0, dstv1]
        xlr = [xlr0, xlr1]
        xrr = [xrr0, xrr1]
        attrv = [attrv0, attrv1]
        dsc = [dsc0, dsc1]
        asc = [asc0, asc1]
        isem = [isem0, isem1]
        gsem = [gsem0, gsem1]
        ssem = [ssem0, ssem1]
        ebase = wid * e_tile

        def _idx_copies(ci, b):
            cb = ebase + ci * CHUNK
            cps = [pltpu.make_async_copy(src_hbm.at[pl.ds(cb, CHUNK)],
                                         srcv[b], isem[b]),
                   pltpu.make_async_copy(dst_hbm.at[pl.ds(cb, CHUNK)],
                                         dstv[b], isem[b])]
            if has_attr:
                cps.append(pltpu.make_async_copy(
                    attr_hbm.at[pl.ds(cb, CHUNK)], attrv[b], isem[b]))
            return cps

        def _gath_copies(b):
            return [pltpu.make_async_copy(xl_hbm.at[srcv[b]], xlr[b], gsem[b]),
                    pltpu.make_async_copy(xr_hbm.at[dstv[b]], xrr[b], gsem[b])]

        def _scat(b):
            return pltpu.make_async_copy(xlr[b], numsh.at[dsc[b]], ssem[b])

        def _zb(i, _):
            for j in range(H // 16):
                xlr0[i, pl.ds(j * 16, 16)] = zero16
            return 0
        lax.fori_loop(0, CHUNK, _zb, 0)

        def _zd(i, _):
            denl[pl.ds(i * 16, 16)] = zero16
            return 0
        lax.fori_loop(0, n_dst // 16, _zd, 0)

        base_rows = s * rows_tile
        for q in range(n_zcopy):
            pltpu.sync_copy(xlr0, numsh.at[pl.ds(base_rows + q * CHUNK, CHUNK)])
        plsc.subcore_barrier()

        pltpu.sync_copy(att_hbm, attv)
        if has_attr:
            pltpu.sync_copy(we_hbm, wev)
        att_j = [attv[pl.ds(j * 16, 16)] for j in range(H // 16)]
        we_j = ([wev[pl.ds(j * 16, 16)] for j in range(H // 16)]
                if has_attr else None)

        for cp in _idx_copies(0, 0):
            cp.start()
        for cp in _idx_copies(1, 1):
            cp.start()
        for cp in _idx_copies(0, 0):
            cp.wait()
        for cp in _gath_copies(0):
            cp.start()

        def _chunk_body(ci, cur):
            nxt = 1 - cur
            for cp in _gath_copies(cur):
                cp.wait()
            for q in range(CHUNK // 16):
                dsc[cur][pl.ds(q * 16, 16)] = dstv[cur][pl.ds(q * 16, 16)]
                if has_attr:
                    asc[cur][pl.ds(q * 16, 16)] = attrv[cur][pl.ds(q * 16, 16)]
            @pl.when(ci + 2 < n_chunks)
            def _():
                for cp in _idx_copies(ci + 2, cur):
                    cp.start()
            @pl.when(ci + 1 < n_chunks)
            def _():
                @pl.when(ci >= 1)
                def _():
                    _scat(nxt).wait()
                for cp in _idx_copies(ci + 1, nxt):
                    cp.wait()
                for cp in _gath_copies(nxt):
                    cp.start()
            lanes = lax.iota(jnp.int32, 16)
            perms = [(lanes ^ sh).reshape(16, 1) for sh in (8, 4, 2, 1)]
            xl_b, xr_b, at_b, ds_b = xlr[cur], xrr[cur], asc[cur], dsc[cur]

            def _dotl(kk, _):
                evv = zero16
                if has_attr:
                    av = at_b[pl.ds(kk * 16, 16)]
                for t in range(16):
                    i = kk * 16 + t
                    acc = zero16
                    xls = []
                    for j in range(H // 16):
                        xlv = xl_b[i, pl.ds(j * 16, 16)]
                        xls.append(xlv)
                        m = xlv + xr_b[i, pl.ds(j * 16, 16)]
                        if has_attr:
                            m = m + av[t] * we_j[j]
                        m = jnp.maximum(m, m * 0.2)
                        acc = acc + m * att_j[j]
                    vvt = jnp.exp(_hsum(acc, perms))
                    sc = vvt[0]
                    for j in range(H // 16):
                        xl_b[i, pl.ds(j * 16, 16)] = xls[j] * sc
                    evv = jnp.where(lanes == t, vvt, evv)
                plsc.addupdate_scatter(denl, [ds_b[pl.ds(kk * 16, 16)]], evv)
                return 0
            lax.fori_loop(0, CHUNK // 16, _dotl, 0)
            _scat(cur).start(add=True)

        def _chunk(ci, _):
            @pl.when(ci % 2 == 0)
            def _():
                _chunk_body(ci, 0)

            @pl.when(ci % 2 == 1)
            def _():
                _chunk_body(ci, 1)
            return 0
        lax.fori_loop(0, n_chunks, _chunk, 0)

        _scat(0).wait()
        _scat(1).wait()

        plsc.subcore_barrier()
        pltpu.sync_copy(denl, den_out.at[c, s])
        for q in range(n_zcopy):
            r0 = base_rows + q * CHUNK
            pltpu.sync_copy(numsh.at[pl.ds(r0, CHUNK)],
                            num_out.at[c].at[pl.ds(r0, CHUNK)])

    return k


def _pad_rows(x, n):
    return jnp.pad(x, ((0, n - x.shape[0]), (0, 0)))


def _pad_edges(src, dst, e_pad, dst_sentinel, attr=None):
    e = src.shape[0]
    src = jnp.pad(src, (0, e_pad - e))
    dst = jnp.pad(dst, (0, e_pad - e), constant_values=dst_sentinel)
    if attr is not None:
        attr = jnp.pad(attr, (0, e_pad - e))
    return src, dst, attr


def _edge_pass(xl, xr, src, dst, att, attr=None, we=None):
    n_src, n_dst = xl.shape[0], xr.shape[0]
    e_pad = src.shape[0]
    has_attr = attr is not None
    if not has_attr:
        attr = jnp.zeros((e_pad,), jnp.float32)
        we = jnp.zeros((H,), jnp.float32)
    k = _edge_kernel(n_src, n_dst, e_pad, has_attr)
    return k(xl, xr, src, dst, att, attr, we)


def _round_up(x, m):
    return (x + m - 1) // m * m


def kernel(params, gene_idx, reaction_idx, metabolite_idx, edge_index_ppi,
           edge_index_reg, edge_index_gpr, edge_index_rmr, edge_attr_rmr,
           gene_batch):
    gn = gene_idx.shape[0]
    rn = reaction_idx.shape[0]
    mn = metabolite_idx.shape[0]
    ngp = _round_up(gn, 1024)
    nrp = _round_up(rn, 1024)
    nmp = _round_up(mn, 1024)

    x_g = _pre_call(_pad_rows(params["emb_gene"], ngp), params["pre"])
    x_r = _pad_rows(params["emb_rxn"], nrp)
    x_m = _pad_rows(params["emb_met"], nmp)

    unit = NW * CHUNK
    e_ppi_pad = _round_up(edge_index_ppi.shape[1], unit)
    e_gpr_pad = _round_up(edge_index_gpr.shape[1], unit)
    src_ppi, dst_ppi, _ = _pad_edges(edge_index_ppi[0], edge_index_ppi[1],
                                     e_ppi_pad, gn)
    src_reg, dst_reg, _ = _pad_edges(edge_index_reg[0], edge_index_reg[1],
                                     e_ppi_pad, gn)
    src_gpr, dst_gpr, _ = _pad_edges(edge_index_gpr[0], edge_index_gpr[1],
                                     e_gpr_pad, rn)
    src_rmr, dst_rmr, attr_rmr = _pad_edges(
        edge_index_rmr[0], edge_index_rmr[1], e_gpr_pad, mn,
        edge_attr_rmr[:, 0])

    for lp in params["layers"]:
        wg = jnp.concatenate([lp["ppi"]["Wl"], lp["ppi"]["Wr"],
                              lp["reg"]["Wl"], lp["reg"]["Wr"],
                              lp["gpr"]["Wl"]], axis=1)
        pg = _mm_call(x_g, wg)
        xl_ppi, xr_ppi = pg[:, 0:H], pg[:, H:2 * H]
        xl_reg, xr_reg = pg[:, 2 * H:3 * H], pg[:, 3 * H:4 * H]
        xl_gpr = pg[:, 4 * H:5 * H]
        wr = jnp.concatenate([lp["gpr"]["Wr"], lp["rmr"]["Wl"]], axis=1)
        pr = _mm_call(x_r, wr)
        xr_gpr, xl_rmr = pr[:, 0:H], pr[:, H:2 * H]
        xr_rmr = _mm_call(x_m, lp["rmr"]["Wr"])

        num_p, den_p = _edge_pass(xl_ppi, xr_ppi, src_ppi, dst_ppi,
                                  lp["ppi"]["att"])
        num_r, den_r = _edge_pass(xl_reg, xr_reg, src_reg, dst_reg,
                                  lp["reg"]["att"])
        num_q, den_q = _edge_pass(xl_gpr, xr_gpr, src_gpr, dst_gpr,
                                  lp["gpr"]["att"])
        num_m, den_m = _edge_pass(xl_rmr, xr_rmr, src_rmr, dst_rmr,
                                  lp["rmr"]["att"], attr_rmr,
                                  lp["rmr"]["We"].reshape(H))

        x_g = _fin2_call(num_p, den_p, lp["ppi"], num_r, den_r, lp["reg"])
        x_r = _fin1_call(num_q, den_q, lp["gpr"])
        x_m = _fin1_call(num_m, den_m, lp["rmr"])

    batch_pad = jnp.pad(gene_batch, (0, ngp - gn), constant_values=4)
    oh = jax.nn.one_hot(batch_pad, H, dtype=jnp.float32)
    num_pool, den_pool = _agg_call(x_g, oh, params["agg"])
    out = _head_call(num_pool, den_pool, params["head"])
    return out[:4, :2]


# speedup vs baseline: 1.5543x; 1.0013x over previous
import functools

import jax
import jax.numpy as jnp
from jax import lax
from jax.experimental import pallas as pl
from jax.experimental.pallas import tpu as pltpu
from jax.experimental.pallas import tpu_sc as plsc

H = 128
NC, NS = 2, 16
NW = NC * NS
CHUNK = 64
RB = 256


def _ln(x, g, b):
    mu = jnp.mean(x, -1, keepdims=True)
    var = jnp.mean((x - mu) ** 2, -1, keepdims=True)
    return (x - mu) / jnp.sqrt(var + 1e-5) * g + b


def _relu(x):
    return jnp.maximum(x, 0.0)


def _dot(a, b):
    return jnp.dot(a, b, preferred_element_type=jnp.float32)


def _full(shape):
    return pl.BlockSpec(shape, lambda i: (0,) * len(shape))


def _pre_body(x_ref, w1, b1, g1, be1, w2, b2, g2, be2, o_ref):
    x = x_ref[...]
    h = _relu(_ln(_dot(x, w1[...]) + b1[...], g1[...], be1[...]))
    o_ref[...] = _relu(_ln(_dot(h, w2[...]) + b2[...], g2[...], be2[...]))


def _pre_call(x, p):
    n = x.shape[0]
    ws = [p["W1"], p["b1"], p["g1"], p["be1"], p["W2"], p["b2"], p["g2"], p["be2"]]
    ws = [w.reshape(1, H) if w.ndim == 1 else w for w in ws]
    specs = [pl.BlockSpec((RB, H), lambda i: (i, 0))]
    specs += [_full(tuple(w.shape)) for w in ws]
    return pl.pallas_call(
        _pre_body,
        grid=(n // RB,),
        in_specs=specs,
        out_specs=pl.BlockSpec((RB, H), lambda i: (i, 0)),
        out_shape=jax.ShapeDtypeStruct((n, H), jnp.float32),
    )(x, *ws)


def _mm_body(x_ref, w_ref, o_ref):
    o_ref[...] = _dot(x_ref[...], w_ref[...])


def _mm_call(x, w):
    n, k = x.shape[0], w.shape[1]
    return pl.pallas_call(
        _mm_body,
        grid=(n // RB,),
        in_specs=[pl.BlockSpec((RB, H), lambda i: (i, 0)), _full((H, k))],
        out_specs=pl.BlockSpec((RB, k), lambda i: (i, 0)),
        out_shape=jax.ShapeDtypeStruct((n, k), jnp.float32),
    )(x, w)


def _fin_part(num_ref, den_ref, b, g, be):
    n = num_ref[0] + num_ref[1]
    d = jnp.sum(den_ref[...], axis=(0, 1))
    y = n / (d[:, None] + 1e-16) + b[...]
    return _relu(_ln(y, g[...], be[...]))


def _fin1_body(num_ref, den_ref, b, g, be, o_ref):
    o_ref[...] = _fin_part(num_ref, den_ref, b, g, be)


def _fin2_body(na, da, ba, ga, bea, nb, db, bb, gb, beb, o_ref):
    o_ref[...] = (_fin_part(na, da, ba, ga, bea)
                  + _fin_part(nb, db, bb, gb, beb))


def _fin_specs(n):
    return [
        pl.BlockSpec((NC, RB, H), lambda i: (0, i, 0)),
        pl.BlockSpec((NC, NS, RB), lambda i: (0, 0, i)),
        _full((1, H)), _full((1, H)), _full((1, H)),
    ]


def _fin1_call(num, den, p):
    n = num.shape[1]
    args = [num, den, p["b"].reshape(1, H), p["ln_g"].reshape(1, H),
            p["ln_b"].reshape(1, H)]
    return pl.pallas_call(
        _fin1_body,
        grid=(n // RB,),
        in_specs=_fin_specs(n),
        out_specs=pl.BlockSpec((RB, H), lambda i: (i, 0)),
        out_shape=jax.ShapeDtypeStruct((n, H), jnp.float32),
    )(*args)


def _fin2_call(numa, dena, pa, numb, denb, pb):
    n = numa.shape[1]
    args = [numa, dena, pa["b"].reshape(1, H), pa["ln_g"].reshape(1, H),
            pa["ln_b"].reshape(1, H),
            numb, denb, pb["b"].reshape(1, H), pb["ln_g"].reshape(1, H),
            pb["ln_b"].reshape(1, H)]
    return pl.pallas_call(
        _fin2_body,
        grid=(n // RB,),
        in_specs=_fin_specs(n) + _fin_specs(n),
        out_specs=pl.BlockSpec((RB, H), lambda i: (i, 0)),
        out_shape=jax.ShapeDtypeStruct((n, H), jnp.float32),
    )(*args)


def _agg_body(x_ref, oh_ref, gw1, gb1, gw2, gb2, tw, tb, num_ref, den_ref):
    i = pl.program_id(0)
    x = x_ref[...]
    h = _relu(_dot(x, gw1[...]) + gb1[...])
    gate = _dot(h, gw2[...]) + gb2[...]
    w = jnp.exp(gate[:, 0:1])
    t = _relu(_dot(x, tw[...]) + tb[...])
    cn = lax.dot_general(oh_ref[...], w * t, (((0,), (0,)), ((), ())),
                         preferred_element_type=jnp.float32)
    cd = lax.dot_general(oh_ref[...], jnp.broadcast_to(w, t.shape),
                         (((0,), (0,)), ((), ())),
                         preferred_element_type=jnp.float32)

    @pl.when(i == 0)
    def _():
        num_ref[...] = cn
        den_ref[...] = cd

    @pl.when(i > 0)
    def _():
        num_ref[...] = num_ref[...] + cn
        den_ref[...] = den_ref[...] + cd


def _agg_call(x, oh, ag):
    n = x.shape[0]
    gw1 = jnp.zeros((H, H), jnp.float32).at[:, : H // 2].set(ag["gW1"])
    gb1 = jnp.zeros((1, H), jnp.float32).at[:, : H // 2].set(ag["gb1"])
    gw2 = jnp.zeros((H, H), jnp.float32).at[: H // 2, 0:1].set(ag["gW2"])
    gb2 = jnp.zeros((1, H), jnp.float32).at[:, 0:1].set(ag["gb2"])
    tb = ag["tb"].reshape(1, H)
    return pl.pallas_call(
        _agg_body,
        grid=(n // RB,),
        in_specs=[pl.BlockSpec((RB, H), lambda i: (i, 0)),
                  pl.BlockSpec((RB, H), lambda i: (i, 0)),
                  _full((H, H)), _full((1, H)), _full((H, H)), _full((1, H)),
                  _full((H, H)), _full((1, H))],
        out_specs=[_full((H, H)), _full((H, H))],
        out_shape=[jax.ShapeDtypeStruct((H, H), jnp.float32),
                   jax.ShapeDtypeStruct((H, H), jnp.float32)],
    )(x, oh, gw1, gb1, gw2, gb2, ag["tW"], tb)


def _head_body(num_ref, den_ref, w, b, o_ref):
    pooled = num_ref[...] / (den_ref[...] + 1e-16)
    o_ref[...] = _dot(pooled, w[...]) + b[...]


def _head_call(num, den, hd):
    w = jnp.zeros((H, H), jnp.float32).at[:, :2].set(hd["W"])
    b = jnp.zeros((1, H), jnp.float32).at[:, :2].set(hd["b"])
    return pl.pallas_call(
        _head_body,
        grid=(1,),
        in_specs=[_full((H, H)), _full((H, H)), _full((H, H)), _full((1, H))],
        out_specs=_full((H, H)),
        out_shape=jax.ShapeDtypeStruct((H, H), jnp.float32),
    )(num, den, w, b)


_GDN = lax.GatherDimensionNumbers(offset_dims=(), collapsed_slice_dims=(0,),
                                  start_index_map=(0,))


def _hsum(a, perms):
    for p in perms:
        g = lax.gather(a, p, _GDN, (1,),
                       mode=lax.GatherScatterMode.PROMISE_IN_BOUNDS)
        a = a + g
    return a


@functools.lru_cache(maxsize=None)
def _edge_kernel(n_src, n_dst, e_pad, has_attr):
    e_tile = e_pad // NW
    n_chunks = e_tile // CHUNK
    rows_tile = n_dst // NS
    n_zcopy = rows_tile // CHUNK
    mesh = plsc.VectorSubcoreMesh(core_axis_name="c", subcore_axis_name="s")

    scratch = [
        pltpu.VMEM((CHUNK,), jnp.int32),
        pltpu.VMEM((CHUNK,), jnp.int32),
        pltpu.VMEM((CHUNK,), jnp.int32),
        pltpu.VMEM((CHUNK,), jnp.int32),
        pltpu.VMEM((CHUNK, H), jnp.float32),
        pltpu.VMEM((CHUNK, H), jnp.float32),
        pltpu.VMEM((CHUNK, H), jnp.float32),
        pltpu.VMEM((CHUNK, H), jnp.float32),
        pltpu.VMEM((CHUNK,), jnp.float32),
        pltpu.VMEM((CHUNK,), jnp.float32),
        pltpu.VMEM((CHUNK,), jnp.int32),
        pltpu.VMEM((CHUNK,), jnp.int32),
        pltpu.VMEM((CHUNK,), jnp.float32),
        pltpu.VMEM((CHUNK,), jnp.float32),
        pltpu.VMEM((n_dst,), jnp.float32),
        pltpu.VMEM((H,), jnp.float32),
        pltpu.VMEM((H,), jnp.float32),
        pltpu.VMEM_SHARED((n_dst, H), jnp.float32),
        pltpu.SemaphoreType.DMA,
        pltpu.SemaphoreType.DMA,
        pltpu.SemaphoreType.DMA,
        pltpu.SemaphoreType.DMA,
        pltpu.SemaphoreType.DMA,
        pltpu.SemaphoreType.DMA,
    ]
    out_type = [jax.ShapeDtypeStruct((NC, n_dst, H), jnp.float32),
                jax.ShapeDtypeStruct((NC, NS, n_dst), jnp.float32)]

    @functools.partial(pl.kernel, mesh=mesh, out_type=out_type,
                       scratch_types=scratch,
                       compiler_params=pltpu.CompilerParams(
                           needs_layout_passes=False))
    def k(xl_hbm, xr_hbm, src_hbm, dst_hbm, att_hbm, attr_hbm, we_hbm,
          num_out, den_out,
          srcv0, srcv1, dstv0, dstv1, xlr0, xlr1, xrr0, xrr1,
          attrv0, attrv1, dsc0, dsc1, asc0, asc1, denl, attv, wev, numsh,
          isem0, isem1, gsem0, gsem1, ssem0, ssem1):
        c = lax.axis_index("c")
        s = lax.axis_index("s")
        wid = c * NS + s
        zero16 = jnp.zeros((16,), jnp.float32)
        srcv = [srcv0, srcv1]
        dstv = [dstv0, dstv1]
        xlr = [xlr0, xlr1]
        xrr = [xrr0, xrr1]
        attrv = [attrv0, attrv1]
        dsc = [dsc0, dsc1]
        asc = [asc0, asc1]
        isem = [isem0, isem1]
        gsem = [gsem0, gsem1]
        ssem = [ssem0, ssem1]
        ebase = wid * e_tile

        def _idx_copies(ci, b):
            cb = ebase + ci * CHUNK
            cps = [pltpu.make_async_copy(src_hbm.at[pl.ds(cb, CHUNK)],
                                         srcv[b], isem[b]),
                   pltpu.make_async_copy(dst_hbm.at[pl.ds(cb, CHUNK)],
                                         dstv[b], isem[b])]
            if has_attr:
                cps.append(pltpu.make_async_copy(
                    attr_hbm.at[pl.ds(cb, CHUNK)], attrv[b], isem[b]))
            return cps

        def _gath_copies(b):
            return [pltpu.make_async_copy(xl_hbm.at[srcv[b]], xlr[b], gsem[b]),
                    pltpu.make_async_copy(xr_hbm.at[dstv[b]], xrr[b], gsem[b])]

        def _scat(b):
            return pltpu.make_async_copy(xlr[b], numsh.at[dsc[b]], ssem[b])

        def _zb(i, _):
            for j in range(H // 16):
                xlr0[i, pl.ds(j * 16, 16)] = zero16
            return 0
        lax.fori_loop(0, CHUNK, _zb, 0)

        def _zd(i, _):
            denl[pl.ds(i * 16, 16)] = zero16
            return 0
        lax.fori_loop(0, n_dst // 16, _zd, 0)

        base_rows = s * rows_tile
        z_cps = [pltpu.make_async_copy(
            xlr0, numsh.at[pl.ds(base_rows + q * CHUNK, CHUNK)], gsem0)
            for q in range(n_zcopy)]
        for cp in z_cps:
            cp.start()
        for cp in z_cps:
            cp.wait()
        plsc.subcore_barrier()

        pltpu.sync_copy(att_hbm, attv)
        if has_attr:
            pltpu.sync_copy(we_hbm, wev)
        att_j = [attv[pl.ds(j * 16, 16)] for j in range(H // 16)]
        we_j = ([wev[pl.ds(j * 16, 16)] for j in range(H // 16)]
                if has_attr else None)

        for cp in _idx_copies(0, 0):
            cp.start()
        for cp in _idx_copies(1, 1):
            cp.start()
        for cp in _idx_copies(0, 0):
            cp.wait()
        for cp in _gath_copies(0):
            cp.start()

        def _chunk_body(ci, cur):
            nxt = 1 - cur
            for cp in _gath_copies(cur):
                cp.wait()
            for q in range(CHUNK // 16):
                dsc[cur][pl.ds(q * 16, 16)] = dstv[cur][pl.ds(q * 16, 16)]
                if has_attr:
                    asc[cur][pl.ds(q * 16, 16)] = attrv[cur][pl.ds(q * 16, 16)]
            @pl.when(ci + 2 < n_chunks)
            def _():
                for cp in _idx_copies(ci + 2, cur):
                    cp.start()
            @pl.when(ci + 1 < n_chunks)
            def _():
                @pl.when(ci >= 1)
                def _():
                    _scat(nxt).wait()
                for cp in _idx_copies(ci + 1, nxt):
                    cp.wait()
                for cp in _gath_copies(nxt):
                    cp.start()
            lanes = lax.iota(jnp.int32, 16)
            perms = [(lanes ^ sh).reshape(16, 1) for sh in (8, 4, 2, 1)]
            xl_b, xr_b, at_b, ds_b = xlr[cur], xrr[cur], asc[cur], dsc[cur]

            def _dotl(kk, _):
                evv = zero16
                if has_attr:
                    av = at_b[pl.ds(kk * 16, 16)]
                for t in range(16):
                    i = kk * 16 + t
                    acc = zero16
                    xls = []
                    for j in range(H // 16):
                        xlv = xl_b[i, pl.ds(j * 16, 16)]
                        xls.append(xlv)
                        m = xlv + xr_b[i, pl.ds(j * 16, 16)]
                        if has_attr:
                            m = m + av[t] * we_j[j]
                        m = jnp.maximum(m, m * 0.2)
                        acc = acc + m * att_j[j]
                    vvt = jnp.exp(_hsum(acc, perms))
                    sc = vvt[0]
                    for j in range(H // 16):
                        xl_b[i, pl.ds(j * 16, 16)] = xls[j] * sc
                    evv = jnp.where(lanes == t, vvt, evv)
                plsc.addupdate_scatter(denl, [ds_b[pl.ds(kk * 16, 16)]], evv)
                return 0
            lax.fori_loop(0, CHUNK // 16, _dotl, 0)
            _scat(cur).start(add=True)

        def _chunk(ci, _):
            @pl.when(ci % 2 == 0)
            def _():
                _chunk_body(ci, 0)

            @pl.when(ci % 2 == 1)
            def _():
                _chunk_body(ci, 1)
            return 0
        lax.fori_loop(0, n_chunks, _chunk, 0)

        _scat(0).wait()
        _scat(1).wait()

        plsc.subcore_barrier()
        pltpu.sync_copy(denl, den_out.at[c, s])
        for q in range(n_zcopy):
            r0 = base_rows + q * CHUNK
            pltpu.sync_copy(numsh.at[pl.ds(r0, CHUNK)],
                            num_out.at[c].at[pl.ds(r0, CHUNK)])

    return k


def _pad_rows(x, n):
    return jnp.pad(x, ((0, n - x.shape[0]), (0, 0)))


def _pad_edges(src, dst, e_pad, dst_sentinel, attr=None):
    e = src.shape[0]
    src = jnp.pad(src, (0, e_pad - e))
    dst = jnp.pad(dst, (0, e_pad - e), constant_values=dst_sentinel)
    if attr is not None:
        attr = jnp.pad(attr, (0, e_pad - e))
    return src, dst, attr


def _edge_pass(xl, xr, src, dst, att, attr=None, we=None):
    n_src, n_dst = xl.shape[0], xr.shape[0]
    e_pad = src.shape[0]
    has_attr = attr is not None
    if not has_attr:
        attr = jnp.zeros((e_pad,), jnp.float32)
        we = jnp.zeros((H,), jnp.float32)
    k = _edge_kernel(n_src, n_dst, e_pad, has_attr)
    return k(xl, xr, src, dst, att, attr, we)


def _round_up(x, m):
    return (x + m - 1) // m * m


def kernel(params, gene_idx, reaction_idx, metabolite_idx, edge_index_ppi,
           edge_index_reg, edge_index_gpr, edge_index_rmr, edge_attr_rmr,
           gene_batch):
    gn = gene_idx.shape[0]
    rn = reaction_idx.shape[0]
    mn = metabolite_idx.shape[0]
    ngp = _round_up(gn, 1024)
    nrp = _round_up(rn, 1024)
    nmp = _round_up(mn, 1024)

    x_g = _pre_call(_pad_rows(params["emb_gene"], ngp), params["pre"])
    x_r = _pad_rows(params["emb_rxn"], nrp)
    x_m = _pad_rows(params["emb_met"], nmp)

    unit = NW * CHUNK
    e_ppi_pad = _round_up(edge_index_ppi.shape[1], unit)
    e_gpr_pad = _round_up(edge_index_gpr.shape[1], unit)
    src_ppi, dst_ppi, _ = _pad_edges(edge_index_ppi[0], edge_index_ppi[1],
                                     e_ppi_pad, gn)
    src_reg, dst_reg, _ = _pad_edges(edge_index_reg[0], edge_index_reg[1],
                                     e_ppi_pad, gn)
    src_gpr, dst_gpr, _ = _pad_edges(edge_index_gpr[0], edge_index_gpr[1],
                                     e_gpr_pad, rn)
    src_rmr, dst_rmr, attr_rmr = _pad_edges(
        edge_index_rmr[0], edge_index_rmr[1], e_gpr_pad, mn,
        edge_attr_rmr[:, 0])

    for lp in params["layers"]:
        wg = jnp.concatenate([lp["ppi"]["Wl"], lp["ppi"]["Wr"],
                              lp["reg"]["Wl"], lp["reg"]["Wr"],
                              lp["gpr"]["Wl"]], axis=1)
        pg = _mm_call(x_g, wg)
        xl_ppi, xr_ppi = pg[:, 0:H], pg[:, H:2 * H]
        xl_reg, xr_reg = pg[:, 2 * H:3 * H], pg[:, 3 * H:4 * H]
        xl_gpr = pg[:, 4 * H:5 * H]
        wr = jnp.concatenate([lp["gpr"]["Wr"], lp["rmr"]["Wl"]], axis=1)
        pr = _mm_call(x_r, wr)
        xr_gpr, xl_rmr = pr[:, 0:H], pr[:, H:2 * H]
        xr_rmr = _mm_call(x_m, lp["rmr"]["Wr"])

        num_p, den_p = _edge_pass(xl_ppi, xr_ppi, src_ppi, dst_ppi,
                                  lp["ppi"]["att"])
        num_r, den_r = _edge_pass(xl_reg, xr_reg, src_reg, dst_reg,
                                  lp["reg"]["att"])
        num_q, den_q = _edge_pass(xl_gpr, xr_gpr, src_gpr, dst_gpr,
                                  lp["gpr"]["att"])
        num_m, den_m = _edge_pass(xl_rmr, xr_rmr, src_rmr, dst_rmr,
                                  lp["rmr"]["att"], attr_rmr,
                                  lp["rmr"]["We"].reshape(H))

        x_g = _fin2_call(num_p, den_p, lp["ppi"], num_r, den_r, lp["reg"])
        x_r = _fin1_call(num_q, den_q, lp["gpr"])
        x_m = _fin1_call(num_m, den_m, lp["rmr"])

    batch_pad = jnp.pad(gene_batch, (0, ngp - gn), constant_values=4)
    oh = jax.nn.one_hot(batch_pad, H, dtype=jnp.float32)
    num_pool, den_pool = _agg_call(x_g, oh, params["agg"])
    out = _head_call(num_pool, den_pool, params["head"])
    return out[:4, :2]
